# baseline (speedup 1.0000x reference)
import functools

import jax
import jax.numpy as jnp
from jax import lax
from jax.experimental import pallas as pl
from jax.experimental.pallas import tpu as pltpu
from jax.experimental.pallas import tpu_sc as plsc

_NC = 2
_NS = 16
_NW = _NC * _NS


def _sc_gather(ids, table, chunk):
    n = ids.shape[0]
    d = table.shape[1]
    per_w = n // _NW
    n_chunks = per_w // chunk
    mesh = plsc.VectorSubcoreMesh(core_axis_name="c", subcore_axis_name="s")

    @functools.partial(
        pl.kernel,
        out_type=jax.ShapeDtypeStruct((n, d), jnp.float32),
        mesh=mesh,
        scratch_types=[
            pltpu.VMEM((chunk,), jnp.int32),
            pltpu.VMEM((chunk, d), jnp.float32),
            pltpu.SemaphoreType.DMA,
        ],
        compiler_params=pltpu.CompilerParams(use_tc_tiling_on_sc=False),
    )
    def k(ids_hbm, table_hbm, out_hbm, idx_v, rows_v, sem):
        wid = lax.axis_index("s") * _NC + lax.axis_index("c")
        base = wid * per_w

        def body(i, carry):
            off = base + i * chunk
            pltpu.sync_copy(ids_hbm.at[pl.ds(off, chunk)], idx_v)
            pltpu.async_copy(table_hbm.at[idx_v], rows_v, sem).wait()
            pltpu.sync_copy(rows_v, out_hbm.at[pl.ds(off, chunk)])
            return carry

        lax.fori_loop(0, n_chunks, body, 0)

    return k(ids, table)


def _tc_layernorm(tok3, pos, gamma, beta, eps=1e-5):
    b, l, e = tok3.shape
    bb = 64

    def body(tok_ref, pos_ref, g_ref, b_ref, o_ref):
        x = tok_ref[...] + pos_ref[...]
        mean = jnp.mean(x, axis=-1, keepdims=True)
        xc = x - mean
        var = jnp.mean(xc * xc, axis=-1, keepdims=True)
        o_ref[...] = xc * lax.rsqrt(var + eps) * g_ref[...] + b_ref[...]

    return pl.pallas_call(
        body,
        grid=(b // bb,),
        in_specs=[
            pl.BlockSpec((bb, l, e), lambda i: (i, 0, 0)),
            pl.BlockSpec((1, l, e), lambda i: (0, 0, 0)),
            pl.BlockSpec((1, 1, e), lambda i: (0, 0, 0)),
            pl.BlockSpec((1, 1, e), lambda i: (0, 0, 0)),
        ],
        out_specs=pl.BlockSpec((bb, l, e), lambda i: (i, 0, 0)),
        out_shape=jax.ShapeDtypeStruct((b, l, e), jnp.float32),
    )(tok3, pos.reshape(1, l, e), gamma.reshape(1, 1, e), beta.reshape(1, 1, e))


def kernel(input_ids, tok_table, pos_table, ln_gamma, ln_beta):
    b, l = input_ids.shape
    e = tok_table.shape[1]
    ids = input_ids.reshape(-1).astype(jnp.int32)
    rows = _sc_gather(ids, tok_table, chunk=800)
    return _tc_layernorm(rows.reshape(b, l, e), pos_table[:l], ln_gamma, ln_beta)
